# Initial kernel scaffold; baseline (speedup 1.0000x reference)
import jax
import jax.numpy as jnp
from jax.experimental import pallas as pl


def kernel(input, W1, b1, W2, b2, W3, b3):
    raise NotImplementedError("write your pallas kernel here")


# speedup vs baseline: 145.9799x; 145.9799x over previous
import functools

import jax
import jax.numpy as jnp
from jax import lax
from jax.experimental import pallas as pl

_THW = 1024
_HW = 256
_FN = 128


def _tc_body(x_ref, a_ref, b_ref, c_ref, bc_ref, out_ref):
    f32 = jnp.float32
    x = x_ref[...]
    xx = x * x
    ones = jnp.ones((1, _FN), f32)
    dn = (((1,), (1,)), ((), ()))
    sq_col = lax.dot_general(xx, ones, dn, preferred_element_type=f32)
    sq_row = lax.dot_general(ones, xx, dn, preferred_element_type=f32)
    g = lax.dot_general(x, x, dn, preferred_element_type=f32)
    d2 = sq_col + sq_row - 2.0 * g
    d2 = jnp.clip(d2, 1e-5, 100.0)

    rio = lax.broadcasted_iota(jnp.int32, (_THW, _THW), 0)
    cio = lax.broadcasted_iota(jnp.int32, (_THW, _THW), 1)
    same_frame = (rio // _HW) == (cio // _HW)
    d2 = jnp.where(same_frame, 1e9, d2)

    r1 = lax.broadcasted_iota(jnp.int32, (_THW, 1), 0)
    c0 = c_ref[0:1, :]
    c1 = c_ref[1:2, :]
    c2 = c_ref[2:3, :]
    in_t = ((r1 // 16) * 4).astype(f32)
    in_h = (r1 % 16).astype(f32)
    p_t = (r1 // _HW).astype(f32) * 0.25
    p_h = ((r1 // 16) % 16).astype(f32)
    p_w = (r1 % 16).astype(f32)
    dn_mm = (((1,), (0,)), ((), ()))
    z = (lax.dot_general(x, a_ref[...], dn_mm, preferred_element_type=f32)
         + bc_ref[...] + in_t * c0 + in_h * c1)
    y = (lax.dot_general(x, b_ref[...], dn_mm, preferred_element_type=f32)
         + p_t * c0 + p_h * c1 + p_w * c2)

    acc = None
    cur = d2
    for _ in range(3):
        m = jnp.min(cur, axis=1, keepdims=True)
        cand = jnp.where(cur == m, cio, 2048)
        ik = jnp.min(cand, axis=1, keepdims=True)
        oh = (cio == ik).astype(f32)
        gk = lax.dot_general(oh, y, (((1,), (0,)), ((), ())),
                             preferred_element_type=f32)
        acc = gk if acc is None else jnp.maximum(acc, gk)
        cur = jnp.where(cio == ik, 1e9, cur)

    out_ref[...] = z + acc


def kernel(input, W1, b1, W2, b2, W3, b3):
    bs, t, fn, h, w = input.shape
    thw = t * h * w
    x = jnp.transpose(input, (0, 1, 3, 4, 2)).reshape(bs * thw, fn)

    M = W2.T @ W3.T
    Wc = W1.T @ M
    A = Wc[:fn]
    B = Wc[fn:2 * fn]
    Cpad = jnp.zeros((8, fn), jnp.float32).at[:3].set(Wc[2 * fn:])
    bc = (b1 @ M + b2 @ W3.T + b3).reshape(1, fn)

    out = pl.pallas_call(
        _tc_body,
        grid=(bs,),
        in_specs=[
            pl.BlockSpec((thw, fn), lambda i: (i, 0)),
            pl.BlockSpec((fn, fn), lambda i: (0, 0)),
            pl.BlockSpec((fn, fn), lambda i: (0, 0)),
            pl.BlockSpec((8, fn), lambda i: (0, 0)),
            pl.BlockSpec((1, fn), lambda i: (0, 0)),
        ],
        out_specs=pl.BlockSpec((thw, fn), lambda i: (i, 0)),
        out_shape=jax.ShapeDtypeStruct((bs * thw, fn), jnp.float32),
    )(x, A, B, Cpad, bc)

    return jnp.transpose(out.reshape(bs, t, h, w, fn), (0, 1, 4, 2, 3))
